# baseline (speedup 1.0000x reference)
import functools

import jax
import jax.numpy as jnp
from jax import lax
from jax.experimental import pallas as pl
from jax.experimental.pallas import tpu as pltpu
from jax.experimental.pallas import tpu_sc as plsc

_LANES = 128
_NC = 2
_NS = 16


@functools.lru_cache(maxsize=None)
def _make_gather(n_rows: int, d: int, gpc: int):
    nw = _NC * _NS
    groups = n_rows // _LANES
    gpw = groups // nw
    n_chunks = gpw // gpc
    rows_pc = gpc * _LANES

    mesh = plsc.VectorSubcoreMesh(core_axis_name="c", subcore_axis_name="s")

    @functools.partial(
        pl.kernel,
        mesh=mesh,
        out_type=jax.ShapeDtypeStruct((n_rows, d), jnp.float32),
        scratch_types=[
            pltpu.VMEM((gpw, _LANES), jnp.int32),
            pltpu.VMEM((rows_pc, d), jnp.float32),
            pltpu.VMEM((rows_pc, d), jnp.float32),
            pltpu.SemaphoreType.DMA,
            pltpu.SemaphoreType.DMA,
            pltpu.SemaphoreType.DMA,
            pltpu.SemaphoreType.DMA,
        ],
        compiler_params=pltpu.CompilerParams(use_tc_tiling_on_sc=False),
    )
    def gather_kernel(table_hbm, idx_hbm, out_hbm, idx_v,
                      rows0, rows1, g0, g1, o0, o1):
        wid = lax.axis_index("s") * _NC + lax.axis_index("c")
        gbase = wid * gpw
        bufs = (rows0, rows1)
        gsems = (g0, g1)
        osems = (o0, o1)

        pltpu.sync_copy(idx_hbm.at[wid], idx_v)

        def fire(ci):
            buf, sem = bufs[ci % 2], gsems[ci % 2]
            return [
                pltpu.async_copy(
                    table_hbm.at[idx_v.at[ci * gpc + g]],
                    buf.at[pl.ds(g * _LANES, _LANES)],
                    sem,
                )
                for g in range(gpc)
            ]

        in_flight = {0: fire(0)}
        out_flight = {}
        for ci in range(n_chunks):
            b = ci % 2
            if ci + 1 < n_chunks:
                if ci - 1 in out_flight:
                    out_flight.pop(ci - 1).wait()
                in_flight[ci + 1] = fire(ci + 1)
            for c in in_flight.pop(ci):
                c.wait()
            out_flight[ci] = pltpu.async_copy(
                bufs[b],
                out_hbm.at[pl.ds((gbase + ci * gpc) * _LANES, rows_pc)],
                osems[b],
            )
        for c in out_flight.values():
            c.wait()

    return gather_kernel


def kernel(table, input_ids):
    b, h = input_ids.shape
    d = table.shape[1]
    n = b * h
    nw = _NC * _NS
    idx = input_ids.reshape(nw, n // (_LANES * nw), _LANES).astype(jnp.int32)
    out = _make_gather(n, d, 5)(table, idx)
    return out.reshape(b, h, d)
